# baseline (speedup 1.0000x reference)
import functools

import jax
import jax.numpy as jnp
from jax import lax
from jax.experimental import pallas as pl
from jax.experimental.pallas import tpu as pltpu
from jax.experimental.pallas import tpu_sc as plsc

N_HEADS = 4
K_EXPERTS = 2
N_EXPERTS = 8
D_IN = 768
D_OUT = 32
HID = 128
N_TOK = 32768

TILE = 4096
LANES = 16


def _bfly_perms(eio):
    return [jnp.bitwise_xor(eio, jnp.full((LANES,), d, jnp.int32))
            for d in (8, 4, 2, 1)]


def _bfly_max_arg(v, idx, perms, vref, iref):
    for perm in perms:
        vref[...] = v
        iref[...] = idx
        ov = plsc.load_gather(vref, [perm])
        oi = plsc.load_gather(iref, [perm])
        take = (ov > v) | ((ov == v) & (oi < idx))
        v = jnp.where(take, ov, v)
        idx = jnp.where(take, oi, idx)
    return v, idx


def _bfly_sum(v, perms, vref):
    for perm in perms:
        vref[...] = v
        v = v + plsc.load_gather(vref, [perm])
    return v


def _sc_loss_body(scal_hbm, loss_hbm, scal_v, loss_v, vv, iv):
    @pl.when((lax.axis_index("c") == 0) & (lax.axis_index("s") == 0))
    def _():
        pltpu.sync_copy(scal_hbm, scal_v)
        eio = lax.iota(jnp.int32, LANES)
        perms = _bfly_perms(eio)
        neg_inf = jnp.full((LANES,), -jnp.inf, dtype=jnp.float32)
        zero = jnp.zeros((LANES,), dtype=jnp.float32)
        one = jnp.full((LANES,), 1.0, dtype=jnp.float32)
        scs = []
        for h in range(N_HEADS):
            s = scal_v[h]
            m1, i1 = _bfly_max_arg(s, eio, perms, vv, iv)
            masked = jnp.where(eio == i1, neg_inf, s)
            m2, i2 = _bfly_max_arg(masked, eio, perms, vv, iv)
            e2 = jnp.exp(m2 - m1)
            denom = e2 + one
            p1 = one / denom
            p2 = e2 / denom
            scs.append(jnp.where(eio == i1, p1, zero)
                       + jnp.where(eio == i2, p2, zero))
        loss = zero
        for a in range(N_HEADS):
            gaa = _bfly_sum(scs[a] * scs[a], perms, vv) - one
            loss = loss + gaa * gaa
            for b in range(a + 1, N_HEADS):
                gab = _bfly_sum(scs[a] * scs[b], perms, vv)
                loss = loss + 2.0 * gab * gab
        loss_v[...] = loss
        pltpu.sync_copy(loss_v, loss_hbm)


def _tc_body(scaling_ref, Wb_ref, bb_ref, W1h_ref, b1_ref, WoT_ref, bo_ref,
             x_ref, out_ref, M_ref, c_ref):
    i = pl.program_id(0)

    @pl.when(i == 0)
    def _prologue():
        s = scaling_ref[...]
        eio = jax.lax.broadcasted_iota(jnp.int32, (N_HEADS, N_EXPERTS), 1)
        m1 = jnp.max(s, axis=1, keepdims=True)
        idx1 = jnp.min(jnp.where(s == m1, eio, N_EXPERTS),
                       axis=1, keepdims=True)
        masked = jnp.where(eio == idx1, -jnp.inf, s)
        m2 = jnp.max(masked, axis=1, keepdims=True)
        idx2 = jnp.min(jnp.where(masked == m2, eio, N_EXPERTS),
                       axis=1, keepdims=True)
        e2 = jnp.exp(m2 - m1)
        denom = 1.0 + e2
        p1 = 1.0 / denom
        p2 = e2 / denom
        scT = (p1 * (eio == idx1).astype(jnp.float32)
               + p2 * (eio == idx2).astype(jnp.float32))
        M = jnp.zeros((D_IN, HID), dtype=jnp.float32)
        c = b1_ref[...]
        for e in range(N_EXPERTS):
            A_e = jnp.zeros((D_OUT, HID), dtype=jnp.float32)
            for h in range(N_HEADS):
                A_e = A_e + scT[h:h + 1, e:e + 1] * W1h_ref[h]
            M = M + jax.lax.dot_general(Wb_ref[e], A_e,
                                        (((1,), (0,)), ((), ())),
                                        preferred_element_type=jnp.float32)
            c = c + jax.lax.dot_general(bb_ref[e:e + 1, :], A_e,
                                        (((1,), (0,)), ((), ())),
                                        preferred_element_type=jnp.float32)
        M_ref[...] = M
        c_ref[...] = c

    xt = x_ref[...]
    z = jax.lax.dot_general(xt, M_ref[...], (((1,), (0,)), ((), ())),
                            preferred_element_type=jnp.float32) + c_ref[...]
    hidden = jnp.maximum(z, 0.0) + jnp.log1p(jnp.exp(-jnp.abs(z)))
    outT = jax.lax.dot_general(WoT_ref[...], hidden,
                               (((1,), (1,)), ((), ())),
                               preferred_element_type=jnp.float32)
    out_ref[0] = outT + bo_ref[...]


@jax.jit
def _run(x, scaling, scaling_pad, Wb, bb, W1h, b1, WoT, bo):
    n = x.shape[0]
    nblk = n // TILE

    sc_loss = functools.partial(
        pl.kernel,
        mesh=plsc.VectorSubcoreMesh(core_axis_name="c", subcore_axis_name="s"),
        out_type=jax.ShapeDtypeStruct((LANES,), jnp.float32),
        scratch_types=[
            pltpu.VMEM((N_HEADS, LANES), jnp.float32),
            pltpu.VMEM((LANES,), jnp.float32),
            pltpu.VMEM((LANES,), jnp.float32),
            pltpu.VMEM((LANES,), jnp.int32),
        ],
        compiler_params=pltpu.CompilerParams(needs_layout_passes=False),
    )(_sc_loss_body)
    loss16 = sc_loss(scaling_pad)

    out3 = pl.pallas_call(
        _tc_body,
        grid=(nblk,),
        in_specs=[
            pl.BlockSpec((N_HEADS, N_EXPERTS), lambda i: (0, 0)),
            pl.BlockSpec((N_EXPERTS, D_IN, D_OUT), lambda i: (0, 0, 0)),
            pl.BlockSpec((N_EXPERTS, D_OUT), lambda i: (0, 0)),
            pl.BlockSpec((N_HEADS, D_OUT, HID), lambda i: (0, 0, 0)),
            pl.BlockSpec((1, HID), lambda i: (0, 0)),
            pl.BlockSpec((1, HID), lambda i: (0, 0)),
            pl.BlockSpec((1, 1), lambda i: (0, 0)),
            pl.BlockSpec((TILE, D_IN), lambda i: (i, 0)),
        ],
        out_specs=pl.BlockSpec((1, 1, TILE), lambda i: (i, 0, 0)),
        out_shape=jax.ShapeDtypeStruct((nblk, 1, TILE), jnp.float32),
        scratch_shapes=[
            pltpu.VMEM((D_IN, HID), jnp.float32),
            pltpu.VMEM((1, HID), jnp.float32),
        ],
        compiler_params=pltpu.CompilerParams(
            dimension_semantics=("arbitrary",)),
    )(scaling, Wb, bb, W1h, b1, WoT, bo, x)
    return out3.reshape(n, 1), loss16[0]


def kernel(x, scaling, Wb, bb, W1, b1, Wo, bo):
    W1h = W1.reshape(D_OUT, N_HEADS, HID).transpose(1, 0, 2)
    scaling_pad = jnp.pad(scaling, ((0, 0), (0, LANES - N_EXPERTS)),
                          constant_values=-jnp.inf)
    return _run(x, scaling, scaling_pad, Wb, bb, W1h, b1.reshape(1, HID),
                Wo.reshape(1, HID), bo.reshape(1, 1))


# speedup vs baseline: 1.0035x; 1.0035x over previous
import functools

import jax
import jax.numpy as jnp
from jax import lax
from jax.experimental import pallas as pl
from jax.experimental.pallas import tpu as pltpu
from jax.experimental.pallas import tpu_sc as plsc

N_HEADS = 4
K_EXPERTS = 2
N_EXPERTS = 8
D_IN = 768
D_OUT = 32
HID = 128
N_TOK = 32768

TILE = 4096
LANES = 16


def _bfly_perms(eio):
    return [jnp.bitwise_xor(eio, jnp.full((LANES,), d, jnp.int32))
            for d in (8, 4, 2, 1)]


def _bfly_max_arg(v, idx, perms, vref, iref):
    for perm in perms:
        vref[...] = v
        iref[...] = idx
        ov = plsc.load_gather(vref, [perm])
        oi = plsc.load_gather(iref, [perm])
        take = (ov > v) | ((ov == v) & (oi < idx))
        v = jnp.where(take, ov, v)
        idx = jnp.where(take, oi, idx)
    return v, idx


def _bfly_sum(v, perms, vref):
    for perm in perms:
        vref[...] = v
        v = v + plsc.load_gather(vref, [perm])
    return v


def _sc_loss_body(scal_hbm, loss_hbm, scal_v, loss_v, vv, iv):
    @pl.when((lax.axis_index("c") == 0) & (lax.axis_index("s") == 0))
    def _():
        pltpu.sync_copy(scal_hbm, scal_v)
        eio = lax.iota(jnp.int32, LANES)
        perms = _bfly_perms(eio)
        neg_inf = jnp.full((LANES,), -jnp.inf, dtype=jnp.float32)
        zero = jnp.zeros((LANES,), dtype=jnp.float32)
        one = jnp.full((LANES,), 1.0, dtype=jnp.float32)
        scs = []
        for h in range(N_HEADS):
            s = scal_v[h]
            m1, i1 = _bfly_max_arg(s, eio, perms, vv, iv)
            masked = jnp.where(eio == i1, neg_inf, s)
            m2, i2 = _bfly_max_arg(masked, eio, perms, vv, iv)
            e2 = jnp.exp(m2 - m1)
            denom = e2 + one
            p1 = one / denom
            p2 = e2 / denom
            scs.append(jnp.where(eio == i1, p1, zero)
                       + jnp.where(eio == i2, p2, zero))
        loss = zero
        for a in range(N_HEADS):
            gaa = _bfly_sum(scs[a] * scs[a], perms, vv) - one
            loss = loss + gaa * gaa
            for b in range(a + 1, N_HEADS):
                gab = _bfly_sum(scs[a] * scs[b], perms, vv)
                loss = loss + 2.0 * gab * gab
        loss_v[...] = loss
        pltpu.sync_copy(loss_v, loss_hbm)


def _tc_body(scaling_ref, Wb_ref, bb_ref, W1h_ref, b1_ref, WoT_ref, bo_ref,
             x_ref, out_ref, M_ref, c_ref):
    i = pl.program_id(0)

    @pl.when(i == 0)
    def _prologue():
        s = scaling_ref[...]
        eio = jax.lax.broadcasted_iota(jnp.int32, (N_HEADS, N_EXPERTS), 1)
        m1 = jnp.max(s, axis=1, keepdims=True)
        idx1 = jnp.min(jnp.where(s == m1, eio, N_EXPERTS),
                       axis=1, keepdims=True)
        masked = jnp.where(eio == idx1, -jnp.inf, s)
        m2 = jnp.max(masked, axis=1, keepdims=True)
        idx2 = jnp.min(jnp.where(masked == m2, eio, N_EXPERTS),
                       axis=1, keepdims=True)
        e2 = jnp.exp(m2 - m1)
        denom = 1.0 + e2
        p1 = 1.0 / denom
        p2 = e2 / denom
        scT = (p1 * (eio == idx1).astype(jnp.float32)
               + p2 * (eio == idx2).astype(jnp.float32))
        M = jnp.zeros((D_IN, HID), dtype=jnp.float32)
        c = b1_ref[...]
        for e in range(N_EXPERTS):
            A_e = jnp.zeros((D_OUT, HID), dtype=jnp.float32)
            for h in range(N_HEADS):
                A_e = A_e + scT[h:h + 1, e:e + 1] * W1h_ref[h]
            M = M + jax.lax.dot_general(Wb_ref[e], A_e,
                                        (((1,), (0,)), ((), ())),
                                        preferred_element_type=jnp.float32)
            c = c + jax.lax.dot_general(bb_ref[e:e + 1, :], A_e,
                                        (((1,), (0,)), ((), ())),
                                        preferred_element_type=jnp.float32)
        M_ref[...] = M
        c_ref[...] = c

    xt = x_ref[...]
    z = jax.lax.dot_general(xt, M_ref[...], (((1,), (0,)), ((), ())),
                            preferred_element_type=jnp.float32) + c_ref[...]
    hidden = jnp.maximum(z, 0.0) + jnp.log1p(jnp.exp(-jnp.abs(z)))
    outT = jax.lax.dot_general(WoT_ref[...], hidden,
                               (((1,), (1,)), ((), ())),
                               preferred_element_type=jnp.float32)
    out_ref[0] = outT + bo_ref[...]


@jax.jit
def _run(x, scaling, scaling_pad, Wb, bb, W1h, b1, WoT, bo):
    n = x.shape[0]
    nblk = n // TILE


    out3 = pl.pallas_call(
        _tc_body,
        grid=(nblk,),
        in_specs=[
            pl.BlockSpec((N_HEADS, N_EXPERTS), lambda i: (0, 0)),
            pl.BlockSpec((N_EXPERTS, D_IN, D_OUT), lambda i: (0, 0, 0)),
            pl.BlockSpec((N_EXPERTS, D_OUT), lambda i: (0, 0)),
            pl.BlockSpec((N_HEADS, D_OUT, HID), lambda i: (0, 0, 0)),
            pl.BlockSpec((1, HID), lambda i: (0, 0)),
            pl.BlockSpec((1, HID), lambda i: (0, 0)),
            pl.BlockSpec((1, 1), lambda i: (0, 0)),
            pl.BlockSpec((TILE, D_IN), lambda i: (i, 0)),
        ],
        out_specs=pl.BlockSpec((1, 1, TILE), lambda i: (i, 0, 0)),
        out_shape=jax.ShapeDtypeStruct((nblk, 1, TILE), jnp.float32),
        scratch_shapes=[
            pltpu.VMEM((D_IN, HID), jnp.float32),
            pltpu.VMEM((1, HID), jnp.float32),
        ],
        compiler_params=pltpu.CompilerParams(
            dimension_semantics=("arbitrary",)),
    )(scaling, Wb, bb, W1h, b1, WoT, bo, x)
    sc_loss = functools.partial(
        pl.kernel,
        mesh=plsc.VectorSubcoreMesh(core_axis_name="c", subcore_axis_name="s"),
        out_type=jax.ShapeDtypeStruct((LANES,), jnp.float32),
        scratch_types=[
            pltpu.VMEM((N_HEADS, LANES), jnp.float32),
            pltpu.VMEM((LANES,), jnp.float32),
            pltpu.VMEM((LANES,), jnp.float32),
            pltpu.VMEM((LANES,), jnp.int32),
        ],
        compiler_params=pltpu.CompilerParams(needs_layout_passes=False),
    )(_sc_loss_body)
    loss16 = sc_loss(scaling_pad)
    return out3.reshape(n, 1), loss16[0]


def kernel(x, scaling, Wb, bb, W1, b1, Wo, bo):
    W1h = W1.reshape(D_OUT, N_HEADS, HID).transpose(1, 0, 2)
    scaling_pad = jnp.pad(scaling, ((0, 0), (0, LANES - N_EXPERTS)),
                          constant_values=-jnp.inf)
    return _run(x, scaling, scaling_pad, Wb, bb, W1h, b1.reshape(1, HID),
                Wo.reshape(1, HID), bo.reshape(1, 1))
